# baseline (speedup 1.0000x reference)
import functools

import jax
import jax.numpy as jnp
from jax import lax
from jax.experimental import pallas as pl
from jax.experimental.pallas import tpu as pltpu
from jax.experimental.pallas import tpu_sc as plsc

L = 16
B = 16
NM = 2
V = 1000000
SRC = 2048
NEG = -3.0e38

BSZ = 256
CK = 8192
NBS = CK // BSZ
GRID = 122
NBTOT = GRID * NBS
COV = GRID * CK
TAIL = V - COV
TAILV = TAIL // L
BROW = BSZ // L


def _merge_sorted(av, ai, bv, bi):
    rv = lax.rev(bv, (0,))
    ri = lax.rev(bi, (0,))
    take = rv > av
    nv = jnp.where(take, rv, av)
    ni = jnp.where(take, ri, ai)
    sv, si = lax.sort((nv, ni), dimension=0, num_keys=1)
    return sv, si


def _merge16(tv, ti, v, pid):
    sv, sid = lax.sort((v, pid), dimension=0, num_keys=1)
    return _merge_sorted(tv, ti, sv, sid)


_GDN = lax.GatherDimensionNumbers(
    offset_dims=(), collapsed_slice_dims=(0,), start_index_map=(0,))


def _bcast0(v):
    zeros = jnp.zeros((L, 1), jnp.int32)
    return lax.gather(v, zeros, _GDN, (1,),
                      mode=lax.GatherScatterMode.PROMISE_IN_BOUNDS)


def _tc_stats():
    def body(lp_ref, wr_ref, attn_ref, bm_ref, aa_ref):
        i = pl.program_id(0)
        x = lp_ref[...]
        v = (x[:, 0, :] + x[:, 1, :]) * 0.5 + wr_ref[...][None, :]
        bm_ref[...] = jnp.max(v.reshape(B, NBS, BSZ), axis=2).reshape(
            1, B, NBS)

        @pl.when(i == 0)
        def _():
            aw = attn_ref[...]
            aa_ref[...] = (aw[:, 0, :] + aw[:, 1, :]) * 0.5

    return pl.pallas_call(
        body,
        grid=(GRID,),
        in_specs=[
            pl.BlockSpec((B, NM, CK), lambda i: (0, 0, i)),
            pl.BlockSpec((CK,), lambda i: (i,)),
            pl.BlockSpec((B, NM, SRC), lambda i: (0, 0, 0)),
        ],
        out_specs=[
            pl.BlockSpec((1, B, NBS), lambda i: (i, 0, 0)),
            pl.BlockSpec((B, SRC), lambda i: (0, 0)),
        ],
        out_shape=(
            jax.ShapeDtypeStruct((GRID, B, NBS), jnp.float32),
            jax.ShapeDtypeStruct((B, SRC), jnp.float32),
        ),
    )


def _tc_select():
    def body(bm_ref, bid_ref):
        x = bm_ref[...]
        bmw = x.transpose(1, 0, 2).reshape(B, NBTOT)
        cols = lax.broadcasted_iota(jnp.int32, (B, NBTOT), 1)
        picks = []
        for _j in range(B):
            am = jnp.argmax(bmw, axis=1)
            picks.append(am)
            bmw = jnp.where(cols == am[:, None], jnp.float32(NEG), bmw)
        bids = jnp.stack(picks, axis=1)
        bid_ref[...] = jnp.concatenate(
            [bids, jnp.zeros((B, 128 - B), jnp.int32)], axis=1)

    return pl.pallas_call(
        body,
        out_shape=jax.ShapeDtypeStruct((B, 128), jnp.int32),
    )


def _tc_gather():
    def body(bids_ref, lp_ref, vg_ref, buf, sem):
        r = pl.program_id(0)
        cps = []
        for j in range(B):
            bid = bids_ref[r * 128 + j]
            off = pl.multiple_of(bid * BSZ, 128)
            cp = pltpu.make_async_copy(
                lp_ref.at[r, :, pl.ds(off, BSZ)], buf.at[j], sem)
            cp.start()
            cps.append(cp)
        for cp in cps:
            cp.wait()
        x = buf[...]
        vg_ref[...] = ((x[:, 0, :] + x[:, 1, :]) * 0.5).reshape(1, B, BSZ)

    return pl.pallas_call(
        body,
        grid_spec=pltpu.PrefetchScalarGridSpec(
            num_scalar_prefetch=1,
            grid=(B,),
            in_specs=[pl.BlockSpec(memory_space=pltpu.MemorySpace.HBM)],
            out_specs=pl.BlockSpec((1, B, BSZ), lambda r, bids: (r, 0, 0)),
            scratch_shapes=[
                pltpu.VMEM((B, NM, BSZ), jnp.float32),
                pltpu.SemaphoreType.DMA,
            ],
        ),
        out_shape=jax.ShapeDtypeStruct((B, B, BSZ), jnp.float32),
    )


def _sc_scan():
    mesh = plsc.VectorSubcoreMesh(core_axis_name="c", subcore_axis_name="s")

    @functools.partial(
        pl.kernel,
        mesh=mesh,
        compiler_params=pltpu.CompilerParams(
            needs_layout_passes=False, use_tc_tiling_on_sc=False),
        out_type=(
            jax.ShapeDtypeStruct((B * L,), jnp.float32),
            jax.ShapeDtypeStruct((B * L,), jnp.int32),
        ),
        scratch_types=[
            pltpu.VMEM((B * BSZ,), jnp.float32),
            pltpu.VMEM((NM * TAIL,), jnp.float32),
            pltpu.VMEM((L,), jnp.int32),
            pltpu.VMEM((B,), jnp.float32),
            pltpu.VMEM((2 * L,), jnp.float32),
            pltpu.VMEM((L,), jnp.float32),
            pltpu.VMEM((L,), jnp.int32),
        ],
    )
    def k3(vg, tailf, bids, wr, prev, cs_out, ci_out,
           vb, tb, bidb, prevb, rwb, stg_s, stg_i):
        c = lax.axis_index("c")
        s = lax.axis_index("s")

        @pl.when(c == 0)
        def _():
            iota = lax.iota(jnp.int32, L)
            negv = jnp.full((L,), NEG, jnp.float32)

            pltpu.sync_copy(vg.at[pl.ds(s * (B * BSZ), B * BSZ)], vb)
            pltpu.sync_copy(tailf.at[pl.ds(s * (NM * TAIL), NM * TAIL)], tb)
            pltpu.sync_copy(bids.at[pl.ds(s * 128, L)], bidb)
            bv = bidb[...]

            pltpu.sync_copy(wr.at[pl.ds(0, 2 * L)], rwb)
            ru = rwb[pl.ds(L, L)]
            r0 = rwb[pl.ds(0, L)]

            pltpu.sync_copy(prev.at[pl.ds(0, B)], prevb)
            pv = prevb[...]
            sv_idx = jnp.zeros((L, 1), jnp.int32) + s
            prev_b = lax.gather(pv, sv_idx, _GDN, (1,),
                                mode=lax.GatherScatterMode.PROMISE_IN_BOUNDS)

            tv = negv
            ti = iota
            t = negv
            for j in range(B):
                bid = bv[j]
                bmask = (jnp.full((L,), 0, jnp.int32) + bid) == 0
                vs = []
                for k in range(BROW):
                    rw = jnp.where(bmask, r0, ru) if k == 0 else ru
                    vs.append(vb[pl.ds(j * BSZ + k * L, L)] + rw)
                gm = vs[0]
                for k in range(1, BROW):
                    gm = jnp.maximum(gm, vs[k])

                def do_merge(args, bid=bid, vs=vs):
                    tv, ti = args
                    for k in range(BROW):
                        def hitk(a2, k=k):
                            tv2, ti2 = a2
                            tok = bid * BSZ + k * L + iota
                            return _merge16(tv2, ti2, vs[k], tok)
                        tv, ti = lax.cond(
                            jnp.any(vs[k] > _bcast0(tv)), hitk,
                            lambda a2: a2, (tv, ti))
                    return tv, ti, _bcast0(tv)

                def skip(args, t=t):
                    tv, ti = args
                    return tv, ti, t

                tv, ti, t = lax.cond(jnp.any(gm > t), do_merge, skip,
                                     (tv, ti))

            for k in range(TAILV):
                a = tb[pl.ds(k * L, L)]
                b = tb[pl.ds(TAIL + k * L, L)]
                v = (a + b) * 0.5 + ru
                tok = COV + k * L + iota

                def hitt(a2, v=v, tok=tok):
                    tv2, ti2 = a2
                    return _merge16(tv2, ti2, v, tok)

                tv, ti = lax.cond(jnp.any(v > t), hitt,
                                  lambda a2: a2, (tv, ti))
                t = _bcast0(tv)

            stg_s[...] = tv + prev_b
            stg_i[...] = (s << 20) | ti
            pltpu.sync_copy(stg_s, cs_out.at[pl.ds(s * L, L)])
            pltpu.sync_copy(stg_i, ci_out.at[pl.ds(s * L, L)])

    return k3


def _merge_kernel():
    mesh = plsc.VectorSubcoreMesh(core_axis_name="c", subcore_axis_name="s")

    @functools.partial(
        pl.kernel,
        mesh=mesh,
        compiler_params=pltpu.CompilerParams(
            needs_layout_passes=False, use_tc_tiling_on_sc=False),
        out_type=(
            jax.ShapeDtypeStruct((B,), jnp.int32),
            jax.ShapeDtypeStruct((B,), jnp.float32),
            jax.ShapeDtypeStruct((B,), jnp.int32),
            jax.ShapeDtypeStruct((B * SRC,), jnp.float32),
        ),
        scratch_types=[
            pltpu.VMEM((B * L,), jnp.float32),
            pltpu.VMEM((B * L,), jnp.int32),
            pltpu.VMEM((L,), jnp.int32),
            pltpu.VMEM((L,), jnp.float32),
            pltpu.VMEM((L,), jnp.int32),
            pltpu.VMEM((SRC,), jnp.float32),
        ],
    )
    def k4(cs, ci, aa, tok_out, sc_out, ph_out, at_out,
           csb, cib, st_t, st_s, st_p, rowb):
        c = lax.axis_index("c")
        s = lax.axis_index("s")

        @pl.when(c == 0)
        def _():
            pltpu.sync_copy(cs, csb)
            pltpu.sync_copy(ci, cib)
            lists = [(csb[pl.ds(w * L, L)], cib[pl.ds(w * L, L)])
                     for w in range(B)]
            while len(lists) > 1:
                lists = [
                    _merge_sorted(*lists[j], *lists[j + 1])
                    for j in range(0, len(lists), 2)
                ]
            fv, fi = lists[0]
            bs = lax.rev(fv, (0,))
            bi = lax.rev(fi, (0,))
            rows = lax.shift_right_logical(bi, 20)

            @pl.when(s == 0)
            def _():
                st_t[...] = jnp.bitwise_and(bi, (1 << 20) - 1)
                st_s[...] = bs
                st_p[...] = rows
                pltpu.sync_copy(st_t, tok_out)
                pltpu.sync_copy(st_s, sc_out)
                pltpu.sync_copy(st_p, ph_out)

            sidx = jnp.zeros((L, 1), jnp.int32) + s
            rsv = lax.gather(rows, sidx, _GDN, (1,),
                             mode=lax.GatherScatterMode.PROMISE_IN_BOUNDS)
            rs = rsv[0] * SRC
            pltpu.sync_copy(aa.at[pl.ds(rs, SRC)], rowb)
            pltpu.sync_copy(rowb, at_out.at[pl.ds(s * SRC, SRC)])

    return k4


def kernel(log_probs, attn_weights, prev_scores, word_rewards):
    bm3, aa = _tc_stats()(log_probs, word_rewards, attn_weights)
    bids_flat = _tc_select()(bm3).reshape(-1)
    vg = _tc_gather()(bids_flat, log_probs)
    tailf = lax.slice(log_probs, (0, 0, COV), (B, NM, V)).reshape(-1)
    cs, ci = _sc_scan()(vg.reshape(-1), tailf, bids_flat,
                        word_rewards, prev_scores)
    toks, scores, hypos, at1 = _merge_kernel()(cs, ci, aa.reshape(-1))
    return toks, scores, hypos, at1.reshape(B, SRC)


# speedup vs baseline: 1.1980x; 1.1980x over previous
import functools

import jax
import jax.numpy as jnp
from jax import lax
from jax.experimental import pallas as pl
from jax.experimental.pallas import tpu as pltpu
from jax.experimental.pallas import tpu_sc as plsc

L = 16
B = 16
NM = 2
V = 1000000
SRC = 2048
NEG = -3.0e38

BSZ = 256
CK = 16384
NBS = CK // BSZ
GRID = 61
NBTOT = GRID * NBS
COV = GRID * CK
TAIL = V - COV
TAILV = TAIL // L
BROW = BSZ // L


def _merge_sorted(av, ai, bv, bi):
    rv = lax.rev(bv, (0,))
    ri = lax.rev(bi, (0,))
    take = rv > av
    nv = jnp.where(take, rv, av)
    ni = jnp.where(take, ri, ai)
    sv, si = lax.sort((nv, ni), dimension=0, num_keys=1)
    return sv, si


def _merge16(tv, ti, v, pid):
    sv, sid = lax.sort((v, pid), dimension=0, num_keys=1)
    return _merge_sorted(tv, ti, sv, sid)


_GDN = lax.GatherDimensionNumbers(
    offset_dims=(), collapsed_slice_dims=(0,), start_index_map=(0,))


def _bcast0(v):
    zeros = jnp.zeros((L, 1), jnp.int32)
    return lax.gather(v, zeros, _GDN, (1,),
                      mode=lax.GatherScatterMode.PROMISE_IN_BOUNDS)


def _tc_stats():
    def body(lp_ref, wr_ref, attn_ref, bm_ref, aa_ref):
        i = pl.program_id(0)
        x = lp_ref[...]
        v = (x[:, 0, :] + x[:, 1, :]) * 0.5 + wr_ref[...][None, :]
        bm_ref[...] = jnp.max(v.reshape(B, NBS, BSZ), axis=2).reshape(
            1, B, NBS)

        @pl.when(i == 0)
        def _():
            aw = attn_ref[...]
            aa_ref[...] = (aw[:, 0, :] + aw[:, 1, :]) * 0.5

    return pl.pallas_call(
        body,
        grid=(GRID,),
        in_specs=[
            pl.BlockSpec((B, NM, CK), lambda i: (0, 0, i)),
            pl.BlockSpec((CK,), lambda i: (i,)),
            pl.BlockSpec((B, NM, SRC), lambda i: (0, 0, 0)),
        ],
        out_specs=[
            pl.BlockSpec((1, B, NBS), lambda i: (i, 0, 0)),
            pl.BlockSpec((B, SRC), lambda i: (0, 0)),
        ],
        out_shape=(
            jax.ShapeDtypeStruct((GRID, B, NBS), jnp.float32),
            jax.ShapeDtypeStruct((B, SRC), jnp.float32),
        ),
    )


def _tc_select():
    def body(bm_ref, bid_ref):
        x = bm_ref[...]
        bmw = x.transpose(1, 0, 2).reshape(B, NBTOT)
        cols = lax.broadcasted_iota(jnp.int32, (B, NBTOT), 1)
        picks = []
        for _j in range(B):
            am = jnp.argmax(bmw, axis=1)
            picks.append(am)
            bmw = jnp.where(cols == am[:, None], jnp.float32(NEG), bmw)
        bids = jnp.stack(picks, axis=1)
        bid_ref[...] = jnp.concatenate(
            [bids, jnp.zeros((B, 128 - B), jnp.int32)], axis=1)

    return pl.pallas_call(
        body,
        out_shape=jax.ShapeDtypeStruct((B, 128), jnp.int32),
    )


def _tc_gather():
    def body(bids_ref, lp_ref, vg_ref, buf, sem):
        r = pl.program_id(0)
        cps = []
        for j in range(B):
            bid = bids_ref[r * 128 + j]
            off = pl.multiple_of(bid * BSZ, 128)
            cp = pltpu.make_async_copy(
                lp_ref.at[r, :, pl.ds(off, BSZ)], buf.at[j], sem)
            cp.start()
            cps.append(cp)
        for cp in cps:
            cp.wait()
        x = buf[...]
        vg_ref[...] = ((x[:, 0, :] + x[:, 1, :]) * 0.5).reshape(1, B, BSZ)

    return pl.pallas_call(
        body,
        grid_spec=pltpu.PrefetchScalarGridSpec(
            num_scalar_prefetch=1,
            grid=(B,),
            in_specs=[pl.BlockSpec(memory_space=pltpu.MemorySpace.HBM)],
            out_specs=pl.BlockSpec((1, B, BSZ), lambda r, bids: (r, 0, 0)),
            scratch_shapes=[
                pltpu.VMEM((B, NM, BSZ), jnp.float32),
                pltpu.SemaphoreType.DMA,
            ],
        ),
        out_shape=jax.ShapeDtypeStruct((B, B, BSZ), jnp.float32),
    )


def _sc_scan():
    mesh = plsc.VectorSubcoreMesh(core_axis_name="c", subcore_axis_name="s")

    @functools.partial(
        pl.kernel,
        mesh=mesh,
        compiler_params=pltpu.CompilerParams(
            needs_layout_passes=False, use_tc_tiling_on_sc=False),
        out_type=(
            jax.ShapeDtypeStruct((B * L,), jnp.float32),
            jax.ShapeDtypeStruct((B * L,), jnp.int32),
        ),
        scratch_types=[
            pltpu.VMEM((B * BSZ,), jnp.float32),
            pltpu.VMEM((NM * TAIL,), jnp.float32),
            pltpu.VMEM((L,), jnp.int32),
            pltpu.VMEM((B,), jnp.float32),
            pltpu.VMEM((2 * L,), jnp.float32),
            pltpu.VMEM((L,), jnp.float32),
            pltpu.VMEM((L,), jnp.int32),
        ],
    )
    def k3(vg, tailf, bids, wr, prev, cs_out, ci_out,
           vb, tb, bidb, prevb, rwb, stg_s, stg_i):
        c = lax.axis_index("c")
        s = lax.axis_index("s")

        @pl.when(c == 0)
        def _():
            iota = lax.iota(jnp.int32, L)
            negv = jnp.full((L,), NEG, jnp.float32)

            pltpu.sync_copy(vg.at[pl.ds(s * (B * BSZ), B * BSZ)], vb)
            pltpu.sync_copy(tailf.at[pl.ds(s * (NM * TAIL), NM * TAIL)], tb)
            pltpu.sync_copy(bids.at[pl.ds(s * 128, L)], bidb)
            bv = bidb[...]

            pltpu.sync_copy(wr.at[pl.ds(0, 2 * L)], rwb)
            ru = rwb[pl.ds(L, L)]
            r0 = rwb[pl.ds(0, L)]

            pltpu.sync_copy(prev.at[pl.ds(0, B)], prevb)
            pv = prevb[...]
            sv_idx = jnp.zeros((L, 1), jnp.int32) + s
            prev_b = lax.gather(pv, sv_idx, _GDN, (1,),
                                mode=lax.GatherScatterMode.PROMISE_IN_BOUNDS)

            tv = negv
            ti = iota
            t = negv
            for j in range(B):
                bid = bv[j]
                bmask = (jnp.full((L,), 0, jnp.int32) + bid) == 0
                vs = []
                for k in range(BROW):
                    rw = jnp.where(bmask, r0, ru) if k == 0 else ru
                    vs.append(vb[pl.ds(j * BSZ + k * L, L)] + rw)
                gm = vs[0]
                for k in range(1, BROW):
                    gm = jnp.maximum(gm, vs[k])

                def do_merge(args, bid=bid, vs=vs):
                    tv, ti = args
                    for k in range(BROW):
                        def hitk(a2, k=k):
                            tv2, ti2 = a2
                            tok = bid * BSZ + k * L + iota
                            return _merge16(tv2, ti2, vs[k], tok)
                        tv, ti = lax.cond(
                            jnp.any(vs[k] > _bcast0(tv)), hitk,
                            lambda a2: a2, (tv, ti))
                    return tv, ti, _bcast0(tv)

                def skip(args, t=t):
                    tv, ti = args
                    return tv, ti, t

                tv, ti, t = lax.cond(jnp.any(gm > t), do_merge, skip,
                                     (tv, ti))

            for k in range(TAILV):
                a = tb[pl.ds(k * L, L)]
                b = tb[pl.ds(TAIL + k * L, L)]
                v = (a + b) * 0.5 + ru
                tok = COV + k * L + iota

                def hitt(a2, v=v, tok=tok):
                    tv2, ti2 = a2
                    return _merge16(tv2, ti2, v, tok)

                tv, ti = lax.cond(jnp.any(v > t), hitt,
                                  lambda a2: a2, (tv, ti))
                t = _bcast0(tv)

            stg_s[...] = tv + prev_b
            stg_i[...] = (s << 20) | ti
            pltpu.sync_copy(stg_s, cs_out.at[pl.ds(s * L, L)])
            pltpu.sync_copy(stg_i, ci_out.at[pl.ds(s * L, L)])

    return k3


def _merge_kernel():
    mesh = plsc.VectorSubcoreMesh(core_axis_name="c", subcore_axis_name="s")

    @functools.partial(
        pl.kernel,
        mesh=mesh,
        compiler_params=pltpu.CompilerParams(
            needs_layout_passes=False, use_tc_tiling_on_sc=False),
        out_type=(
            jax.ShapeDtypeStruct((B,), jnp.int32),
            jax.ShapeDtypeStruct((B,), jnp.float32),
            jax.ShapeDtypeStruct((B,), jnp.int32),
            jax.ShapeDtypeStruct((B * SRC,), jnp.float32),
        ),
        scratch_types=[
            pltpu.VMEM((B * L,), jnp.float32),
            pltpu.VMEM((B * L,), jnp.int32),
            pltpu.VMEM((L,), jnp.int32),
            pltpu.VMEM((L,), jnp.float32),
            pltpu.VMEM((L,), jnp.int32),
            pltpu.VMEM((SRC,), jnp.float32),
        ],
    )
    def k4(cs, ci, aa, tok_out, sc_out, ph_out, at_out,
           csb, cib, st_t, st_s, st_p, rowb):
        c = lax.axis_index("c")
        s = lax.axis_index("s")

        @pl.when(c == 0)
        def _():
            pltpu.sync_copy(cs, csb)
            pltpu.sync_copy(ci, cib)
            lists = [(csb[pl.ds(w * L, L)], cib[pl.ds(w * L, L)])
                     for w in range(B)]
            while len(lists) > 1:
                lists = [
                    _merge_sorted(*lists[j], *lists[j + 1])
                    for j in range(0, len(lists), 2)
                ]
            fv, fi = lists[0]
            bs = lax.rev(fv, (0,))
            bi = lax.rev(fi, (0,))
            rows = lax.shift_right_logical(bi, 20)

            @pl.when(s == 0)
            def _():
                st_t[...] = jnp.bitwise_and(bi, (1 << 20) - 1)
                st_s[...] = bs
                st_p[...] = rows
                pltpu.sync_copy(st_t, tok_out)
                pltpu.sync_copy(st_s, sc_out)
                pltpu.sync_copy(st_p, ph_out)

            sidx = jnp.zeros((L, 1), jnp.int32) + s
            rsv = lax.gather(rows, sidx, _GDN, (1,),
                             mode=lax.GatherScatterMode.PROMISE_IN_BOUNDS)
            rs = rsv[0] * SRC
            pltpu.sync_copy(aa.at[pl.ds(rs, SRC)], rowb)
            pltpu.sync_copy(rowb, at_out.at[pl.ds(s * SRC, SRC)])

    return k4


def kernel(log_probs, attn_weights, prev_scores, word_rewards):
    bm3, aa = _tc_stats()(log_probs, word_rewards, attn_weights)
    bids_flat = _tc_select()(bm3).reshape(-1)
    vg = _tc_gather()(bids_flat, log_probs)
    tailf = lax.slice(log_probs, (0, 0, COV), (B, NM, V)).reshape(-1)
    cs, ci = _sc_scan()(vg.reshape(-1), tailf, bids_flat,
                        word_rewards, prev_scores)
    toks, scores, hypos, at1 = _merge_kernel()(cs, ci, aa.reshape(-1))
    return toks, scores, hypos, at1.reshape(B, SRC)
